# baseline (speedup 1.0000x reference)
import functools

import jax
import jax.numpy as jnp
from jax import lax
from jax.experimental import pallas as pl
from jax.experimental.pallas import tpu as pltpu
from jax.experimental.pallas import tpu_sc as plsc

NUM_CORES = 2
NUM_SUBCORES = 16
LANES = 16
NUM_WORKERS = NUM_CORES * NUM_SUBCORES

BATCH = 16384
EMBED = 16
NROWS = 1000000
B_PER_W = BATCH // NUM_WORKERS
CHUNK = 128
NCHUNKS = B_PER_W // CHUNK
NGROUPS = B_PER_W // LANES

TW = 7936
TSTEPS = -(-NROWS // TW)
TLAST = NROWS - (TSTEPS - 1) * TW


def _t_body(in_ref, out_hbm, xt_ref, sem):
    j = pl.program_id(0)
    xt_ref[...] = in_ref[...].T

    @pl.when(j < TSTEPS - 1)
    def _full():
        pltpu.async_copy(
            xt_ref, out_hbm.at[pl.ds(j * TW, TW), :], sem).wait()

    @pl.when(j == TSTEPS - 1)
    def _partial():
        pltpu.async_copy(
            xt_ref.at[pl.ds(0, TLAST), :],
            out_hbm.at[pl.ds(j * TW, TLAST), :], sem).wait()


_tc_relayout = pl.pallas_call(
    _t_body,
    grid=(TSTEPS,),
    in_specs=[pl.BlockSpec((EMBED, TW), lambda j: (0, j))],
    out_specs=pl.BlockSpec(memory_space=pl.ANY),
    out_shape=jax.ShapeDtypeStruct((NROWS, EMBED), jnp.float32),
    scratch_shapes=[
        pltpu.VMEM((TW, EMBED), jnp.float32),
        pltpu.SemaphoreType.DMA,
    ],
)


def _b_body(in_ref, out_ref):
    out_ref[...] = in_ref[0, :]


BW = 8192
BSTEPS = -(-NROWS // BW)

_tc_bias_linearize = pl.pallas_call(
    _b_body,
    grid=(BSTEPS,),
    in_specs=[pl.BlockSpec((1, BW), lambda j: (0, j))],
    out_specs=pl.BlockSpec((BW,), lambda j: (j,)),
    out_shape=jax.ShapeDtypeStruct((NROWS,), jnp.float32),
)


def _vperm(v, idx):
    return jnp.take_along_axis(v, idx, axis=0)


def _body(uid_hbm, iid_hbm, uemb_hbm, iemb_hbm, ubias_hbm, ibias_hbm,
          out_hbm, uidx_v, iidx_v, urows_v, irows_v, ub_v, ib_v, out_v,
          sem):
    wid = lax.axis_index("s") * NUM_CORES + lax.axis_index("c")
    base = wid * B_PER_W

    for j in range(NCHUNKS):
        pltpu.sync_copy(uid_hbm.at[pl.ds(base + j * CHUNK, CHUNK)],
                        uidx_v.at[j])
        pltpu.sync_copy(iid_hbm.at[pl.ds(base + j * CHUNK, CHUNK)],
                        iidx_v.at[j])

    copies = []
    for j in range(NCHUNKS):
        lo = j * CHUNK
        copies.append(pltpu.async_copy(
            uemb_hbm.at[uidx_v.at[j]], urows_v.at[pl.ds(lo, CHUNK)], sem))
        copies.append(pltpu.async_copy(
            iemb_hbm.at[iidx_v.at[j]], irows_v.at[pl.ds(lo, CHUNK)], sem))
        copies.append(pltpu.async_copy(
            ubias_hbm.at[uidx_v.at[j]], ub_v.at[pl.ds(lo, CHUNK)], sem))
        copies.append(pltpu.async_copy(
            ibias_hbm.at[iidx_v.at[j]], ib_v.at[pl.ds(lo, CHUNK)], sem))
    for c in copies:
        c.wait()

    iota = lax.iota(jnp.int32, LANES)
    bitrev = (((iota & 1) << 3) | ((iota & 2) << 1)
              | ((iota & 4) >> 1) | ((iota & 8) >> 3))
    masks = {h: (iota & h) == 0 for h in (8, 4, 2, 1)}
    perms = {h: iota ^ h for h in (8, 4, 2, 1)}

    def combine(a, b, h):
        r1 = jnp.where(masks[h], a, b)
        r2 = jnp.where(masks[h], b, a)
        return r1 + _vperm(r2, perms[h])

    def group(g, carry):
        base_r = g * LANES
        vecs = []
        for j in range(LANES):
            u = urows_v[base_r + j]
            v = irows_v[base_r + j]
            vecs.append(u * v)
        for h in (8, 4, 2, 1):
            vecs = [combine(vecs[2 * t], vecs[2 * t + 1], h)
                    for t in range(len(vecs) // 2)]
        z = vecs[0]
        ridx = base_r + bitrev
        ub = plsc.load_gather(ub_v, [ridx])
        ib = plsc.load_gather(ib_v, [ridx])
        x = z + ub + ib
        x = jnp.maximum(x, 0.0)
        e2 = jnp.exp(-2.0 * x)
        y = (1.0 - e2) / (1.0 + e2)
        plsc.store_scatter(out_v, [ridx], y)
        return carry

    lax.fori_loop(0, NGROUPS, group, 0)

    pltpu.sync_copy(out_v, out_hbm.at[pl.ds(base, B_PER_W)])


_sc_kernel = pl.kernel(
    _body,
    out_type=jax.ShapeDtypeStruct((BATCH,), jnp.float32),
    mesh=plsc.VectorSubcoreMesh(core_axis_name="c", subcore_axis_name="s"),
    compiler_params=pltpu.CompilerParams(
        needs_layout_passes=False, use_tc_tiling_on_sc=False),
    scratch_types=[
        pltpu.VMEM((NCHUNKS, CHUNK), jnp.int32),
        pltpu.VMEM((NCHUNKS, CHUNK), jnp.int32),
        pltpu.VMEM((B_PER_W, EMBED), jnp.float32),
        pltpu.VMEM((B_PER_W, EMBED), jnp.float32),
        pltpu.VMEM((B_PER_W,), jnp.float32),
        pltpu.VMEM((B_PER_W,), jnp.float32),
        pltpu.VMEM((B_PER_W,), jnp.float32),
        pltpu.SemaphoreType.DMA,
    ],
)


@jax.jit
def kernel(inputs, user_emb, item_emb, user_bias, item_bias):
    uid = inputs[:, 0].astype(jnp.int32)
    iid = inputs[:, 1].astype(jnp.int32)
    uemb_rm = _tc_relayout(user_emb.T)
    iemb_rm = _tc_relayout(item_emb.T)
    ub1 = _tc_bias_linearize(user_bias.T)
    ib1 = _tc_bias_linearize(item_bias.T)
    out = _sc_kernel(uid, iid, uemb_rm, iemb_rm, ub1, ib1)
    return out.reshape(BATCH, 1)


# speedup vs baseline: 1.8755x; 1.8755x over previous
import jax
import jax.numpy as jnp
from jax import lax
from jax.experimental import pallas as pl
from jax.experimental.pallas import tpu as pltpu
from jax.experimental.pallas import tpu_sc as plsc

NUM_CORES = 2
NUM_SUBCORES = 16
LANES = 16
NUM_WORKERS = NUM_CORES * NUM_SUBCORES

BATCH = 16384
EMBED = 16
NROWS = 1000000
RPG = 128 // EMBED
GROWS = NROWS // RPG
B_PER_W = BATCH // NUM_WORKERS
CHUNK = 128
NCHUNKS = B_PER_W // CHUNK
GPC = CHUNK // LANES


def _vperm(v, idx):
    return jnp.take_along_axis(v, idx, axis=0)


def _body(uid_hbm, iid_hbm, uemb_hbm, iemb_hbm, ubias_hbm, ibias_hbm,
          out_hbm, uidx_v, iidx_v, urow_v, irow_v, u_buf0, u_buf1,
          i_buf0, i_buf1, ub_v, ib_v, out_v, sem):
    wid = lax.axis_index("s") * NUM_CORES + lax.axis_index("c")
    base = wid * B_PER_W
    u_bufs = (u_buf0, u_buf1)
    i_bufs = (i_buf0, i_buf1)

    for j in range(NCHUNKS):
        pltpu.sync_copy(uid_hbm.at[pl.ds(base + j * CHUNK, CHUNK)],
                        uidx_v.at[j])
        pltpu.sync_copy(iid_hbm.at[pl.ds(base + j * CHUNK, CHUNK)],
                        iidx_v.at[j])
    for j in range(NCHUNKS):
        for k in range(CHUNK // LANES):
            sl = pl.ds(k * LANES, LANES)
            urow_v[j, sl] = lax.shift_right_logical(uidx_v[j, sl], 3)
            irow_v[j, sl] = lax.shift_right_logical(iidx_v[j, sl], 3)

    bias_copies = []
    for j in range(NCHUNKS):
        bias_copies.append(pltpu.async_copy(
            ubias_hbm.at[uidx_v.at[j]],
            ub_v.at[pl.ds(j * CHUNK, CHUNK)], sem))
        bias_copies.append(pltpu.async_copy(
            ibias_hbm.at[iidx_v.at[j]],
            ib_v.at[pl.ds(j * CHUNK, CHUNK)], sem))

    def fire(j):
        return (pltpu.async_copy(uemb_hbm.at[urow_v.at[j]],
                                 u_bufs[j % 2], sem),
                pltpu.async_copy(iemb_hbm.at[irow_v.at[j]],
                                 i_bufs[j % 2], sem))

    inflight = fire(0)
    for c in bias_copies:
        c.wait()

    iota = lax.iota(jnp.int32, LANES)
    bitrev = (((iota & 1) << 3) | ((iota & 2) << 1)
              | ((iota & 4) >> 1) | ((iota & 8) >> 3))
    masks = {h: (iota & h) == 0 for h in (8, 4, 2, 1)}
    perms = {h: iota ^ h for h in (8, 4, 2, 1)}

    def combine(a, b, h):
        r1 = jnp.where(masks[h], a, b)
        r2 = jnp.where(masks[h], b, a)
        return r1 + _vperm(r2, perms[h])

    for j in range(NCHUNKS):
        for c in inflight:
            c.wait()
        if j + 1 < NCHUNKS:
            inflight = fire(j + 1)
        ubuf = u_bufs[j % 2]
        ibuf = i_bufs[j % 2]

        def group(g, carry, j=j, ubuf=ubuf, ibuf=ibuf):
            k = g * LANES
            uoffs = (uidx_v[j, pl.ds(k, LANES)] & 7) << 4
            ioffs = (iidx_v[j, pl.ds(k, LANES)] & 7) << 4
            vecs = []
            for lane in range(LANES):
                u = ubuf[k + lane, pl.ds(uoffs[lane], EMBED)]
                v = ibuf[k + lane, pl.ds(ioffs[lane], EMBED)]
                vecs.append(u * v)
            for h in (8, 4, 2, 1):
                vecs = [combine(vecs[2 * t], vecs[2 * t + 1], h)
                        for t in range(len(vecs) // 2)]
            z = vecs[0]
            ridx = j * CHUNK + k + bitrev
            ubv = plsc.load_gather(ub_v, [ridx])
            ibv = plsc.load_gather(ib_v, [ridx])
            x = z + ubv + ibv
            x = jnp.maximum(x, 0.0)
            e2 = jnp.exp(-2.0 * x)
            y = (1.0 - e2) / (1.0 + e2)
            plsc.store_scatter(out_v, [ridx], y)
            return carry

        lax.fori_loop(0, GPC, group, 0)

    pltpu.sync_copy(out_v, out_hbm.at[pl.ds(base, B_PER_W)])


_sc_kernel = pl.kernel(
    _body,
    out_type=jax.ShapeDtypeStruct((BATCH,), jnp.float32),
    mesh=plsc.VectorSubcoreMesh(core_axis_name="c", subcore_axis_name="s"),
    compiler_params=pltpu.CompilerParams(
        needs_layout_passes=False, use_tc_tiling_on_sc=False),
    scratch_types=[
        pltpu.VMEM((NCHUNKS, CHUNK), jnp.int32),
        pltpu.VMEM((NCHUNKS, CHUNK), jnp.int32),
        pltpu.VMEM((NCHUNKS, CHUNK), jnp.int32),
        pltpu.VMEM((NCHUNKS, CHUNK), jnp.int32),
        pltpu.VMEM((CHUNK, 128), jnp.float32),
        pltpu.VMEM((CHUNK, 128), jnp.float32),
        pltpu.VMEM((CHUNK, 128), jnp.float32),
        pltpu.VMEM((CHUNK, 128), jnp.float32),
        pltpu.VMEM((B_PER_W,), jnp.float32),
        pltpu.VMEM((B_PER_W,), jnp.float32),
        pltpu.VMEM((B_PER_W,), jnp.float32),
        pltpu.SemaphoreType.DMA,
    ],
)


@jax.jit
def kernel(inputs, user_emb, item_emb, user_bias, item_bias):
    uid = inputs[:, 0].astype(jnp.int32)
    iid = inputs[:, 1].astype(jnp.int32)
    out = _sc_kernel(uid, iid,
                     user_emb.reshape(GROWS, 128),
                     item_emb.reshape(GROWS, 128),
                     user_bias.reshape(-1), item_bias.reshape(-1))
    return out.reshape(BATCH, 1)


# speedup vs baseline: 2.2239x; 1.1858x over previous
import jax
import jax.numpy as jnp
from jax import lax
from jax.experimental import pallas as pl
from jax.experimental.pallas import tpu as pltpu
from jax.experimental.pallas import tpu_sc as plsc

NUM_CORES = 2
NUM_SUBCORES = 16
LANES = 16
NUM_WORKERS = NUM_CORES * NUM_SUBCORES

BATCH = 16384
EMBED = 16
NROWS = 1000000
RPG = 8
B_PER_W = BATCH // NUM_WORKERS
CHUNK = 128
NCHUNKS = B_PER_W // CHUNK
NGROUPS = B_PER_W // LANES


def _vperm(v, idx):
    return jnp.take_along_axis(v, idx, axis=0)


def _body(uid_hbm, iid_hbm, uemb_hbm, iemb_hbm, ubias_hbm, ibias_hbm,
          out_hbm, uidx_v, iidx_v, u_buf, i_buf, ub_v, ib_v, out_v, sem):
    wid = lax.axis_index("s") * NUM_CORES + lax.axis_index("c")
    base = wid * B_PER_W

    for j in range(NCHUNKS):
        pltpu.sync_copy(uid_hbm.at[pl.ds(base + j * CHUNK, CHUNK)],
                        uidx_v.at[j])
        pltpu.sync_copy(iid_hbm.at[pl.ds(base + j * CHUNK, CHUNK)],
                        iidx_v.at[j])

    bias_copies = []
    for j in range(NCHUNKS):
        bias_copies.append(pltpu.async_copy(
            ubias_hbm.at[uidx_v.at[j]],
            ub_v.at[pl.ds(j * CHUNK, CHUNK)], sem))
        bias_copies.append(pltpu.async_copy(
            ibias_hbm.at[iidx_v.at[j]],
            ib_v.at[pl.ds(j * CHUNK, CHUNK)], sem))
    for c in bias_copies:
        c.wait()

    iota = lax.iota(jnp.int32, LANES)
    bitrev = (((iota & 1) << 3) | ((iota & 2) << 1)
              | ((iota & 4) >> 1) | ((iota & 8) >> 3))
    masks = {h: (iota & h) == 0 for h in (8, 4, 2, 1)}
    perms = {h: iota ^ h for h in (8, 4, 2, 1)}

    def combine(a, b, h):
        r1 = jnp.where(masks[h], a, b)
        r2 = jnp.where(masks[h], b, a)
        return r1 + _vperm(r2, perms[h])

    def group(g, carry):
        j = g // (CHUNK // LANES)
        k = (g - j * (CHUNK // LANES)) * LANES
        uvec = uidx_v[j, pl.ds(k, LANES)]
        ivec = iidx_v[j, pl.ds(k, LANES)]
        copies = []
        for lane in range(LANES):
            ur = pl.multiple_of((uvec[lane] >> 3) << 3, 8)
            ir = pl.multiple_of((ivec[lane] >> 3) << 3, 8)
            copies.append(pltpu.async_copy(
                uemb_hbm.at[pl.ds(ur, RPG), :], u_buf.at[lane], sem))
            copies.append(pltpu.async_copy(
                iemb_hbm.at[pl.ds(ir, RPG), :], i_buf.at[lane], sem))
        for c in copies:
            c.wait()
        usub = uvec & 7
        isub = ivec & 7
        vecs = []
        for lane in range(LANES):
            u = u_buf[lane, usub[lane], :]
            v = i_buf[lane, isub[lane], :]
            vecs.append(u * v)
        for h in (8, 4, 2, 1):
            vecs = [combine(vecs[2 * t], vecs[2 * t + 1], h)
                    for t in range(len(vecs) // 2)]
        z = vecs[0]
        ridx = g * LANES + bitrev
        ubv = plsc.load_gather(ub_v, [ridx])
        ibv = plsc.load_gather(ib_v, [ridx])
        x = z + ubv + ibv
        x = jnp.maximum(x, 0.0)
        e2 = jnp.exp(-2.0 * x)
        y = (1.0 - e2) / (1.0 + e2)
        plsc.store_scatter(out_v, [ridx], y)
        return carry

    lax.fori_loop(0, NGROUPS, group, 0)

    pltpu.sync_copy(out_v, out_hbm.at[pl.ds(base, B_PER_W)])


_sc_kernel = pl.kernel(
    _body,
    out_type=jax.ShapeDtypeStruct((BATCH,), jnp.float32),
    mesh=plsc.VectorSubcoreMesh(core_axis_name="c", subcore_axis_name="s"),
    compiler_params=pltpu.CompilerParams(
        needs_layout_passes=False, use_tc_tiling_on_sc=True),
    scratch_types=[
        pltpu.VMEM((NCHUNKS, CHUNK), jnp.int32),
        pltpu.VMEM((NCHUNKS, CHUNK), jnp.int32),
        pltpu.VMEM((LANES, RPG, EMBED), jnp.float32),
        pltpu.VMEM((LANES, RPG, EMBED), jnp.float32),
        pltpu.VMEM((B_PER_W,), jnp.float32),
        pltpu.VMEM((B_PER_W,), jnp.float32),
        pltpu.VMEM((B_PER_W,), jnp.float32),
        pltpu.SemaphoreType.DMA,
    ],
)


@jax.jit
def kernel(inputs, user_emb, item_emb, user_bias, item_bias):
    uid = inputs[:, 0].astype(jnp.int32)
    iid = inputs[:, 1].astype(jnp.int32)
    out = _sc_kernel(uid, iid, user_emb, item_emb,
                     user_bias.reshape(-1), item_bias.reshape(-1))
    return out.reshape(BATCH, 1)


# speedup vs baseline: 6.8510x; 3.0806x over previous
import jax
import jax.numpy as jnp
from jax import lax
from jax.experimental import pallas as pl
from jax.experimental.pallas import tpu as pltpu
from jax.experimental.pallas import tpu_sc as plsc

NUM_CORES = 2
NUM_SUBCORES = 16
LANES = 16
NUM_WORKERS = NUM_CORES * NUM_SUBCORES

BATCH = 16384
EMBED = 16
NROWS = 1000000
WIN = 128
B_PER_W = BATCH // NUM_WORKERS
CHUNK = 128
NCHUNKS = B_PER_W // CHUNK
NGROUPS = B_PER_W // LANES


def _body(uid_hbm, iid_hbm, uembT_hbm, iembT_hbm, ubias_hbm, ibias_hbm,
          out_hbm, uidx_v, iidx_v, u_buf, i_buf, ub_v, ib_v, out_v, sem):
    wid = lax.axis_index("s") * NUM_CORES + lax.axis_index("c")
    base = wid * B_PER_W

    for j in range(NCHUNKS):
        pltpu.sync_copy(uid_hbm.at[pl.ds(base + j * CHUNK, CHUNK)],
                        uidx_v.at[j])
        pltpu.sync_copy(iid_hbm.at[pl.ds(base + j * CHUNK, CHUNK)],
                        iidx_v.at[j])

    bias_copies = []
    for j in range(NCHUNKS):
        bias_copies.append(pltpu.async_copy(
            ubias_hbm.at[uidx_v.at[j]],
            ub_v.at[pl.ds(j * CHUNK, CHUNK)], sem))
        bias_copies.append(pltpu.async_copy(
            ibias_hbm.at[iidx_v.at[j]],
            ib_v.at[pl.ds(j * CHUNK, CHUNK)], sem))
    for c in bias_copies:
        c.wait()

    iota = lax.iota(jnp.int32, LANES)

    def group(g, carry):
        j = g // (CHUNK // LANES)
        k = (g - j * (CHUNK // LANES)) * LANES
        uvec = uidx_v[j, pl.ds(k, LANES)]
        ivec = iidx_v[j, pl.ds(k, LANES)]
        copies = []
        for lane in range(LANES):
            ub_ = pl.multiple_of((uvec[lane] >> 7) << 7, WIN)
            ib_ = pl.multiple_of((ivec[lane] >> 7) << 7, WIN)
            copies.append(pltpu.async_copy(
                uembT_hbm.at[:, pl.ds(ub_, WIN)],
                u_buf.at[:, pl.ds(lane * WIN, WIN)], sem))
            copies.append(pltpu.async_copy(
                iembT_hbm.at[:, pl.ds(ib_, WIN)],
                i_buf.at[:, pl.ds(lane * WIN, WIN)], sem))
        for c in copies:
            c.wait()

        lane_u = iota * WIN + (uvec & (WIN - 1))
        lane_i = iota * WIN + (ivec & (WIN - 1))
        acc = jnp.zeros((LANES,), jnp.float32)
        for e in range(EMBED):
            ev = jnp.full((LANES,), e, jnp.int32)
            u = plsc.load_gather(u_buf, [ev, lane_u])
            v = plsc.load_gather(i_buf, [ev, lane_i])
            acc = acc + u * v
        x = acc + ub_v[pl.ds(g * LANES, LANES)] + ib_v[pl.ds(g * LANES, LANES)]
        x = jnp.maximum(x, 0.0)
        e2 = jnp.exp(-2.0 * x)
        out_v[pl.ds(g * LANES, LANES)] = (1.0 - e2) / (1.0 + e2)
        return carry

    lax.fori_loop(0, NGROUPS, group, 0)

    pltpu.sync_copy(out_v, out_hbm.at[pl.ds(base, B_PER_W)])


_sc_kernel = pl.kernel(
    _body,
    out_type=jax.ShapeDtypeStruct((BATCH,), jnp.float32),
    mesh=plsc.VectorSubcoreMesh(core_axis_name="c", subcore_axis_name="s"),
    compiler_params=pltpu.CompilerParams(
        needs_layout_passes=False, use_tc_tiling_on_sc=True),
    scratch_types=[
        pltpu.VMEM((NCHUNKS, CHUNK), jnp.int32),
        pltpu.VMEM((NCHUNKS, CHUNK), jnp.int32),
        pltpu.VMEM((EMBED, LANES * WIN), jnp.float32),
        pltpu.VMEM((EMBED, LANES * WIN), jnp.float32),
        pltpu.VMEM((B_PER_W,), jnp.float32),
        pltpu.VMEM((B_PER_W,), jnp.float32),
        pltpu.VMEM((B_PER_W,), jnp.float32),
        pltpu.SemaphoreType.DMA,
    ],
)


@jax.jit
def kernel(inputs, user_emb, item_emb, user_bias, item_bias):
    uid = inputs[:, 0].astype(jnp.int32)
    iid = inputs[:, 1].astype(jnp.int32)
    out = _sc_kernel(uid, iid, user_emb.T, item_emb.T,
                     user_bias.reshape(-1), item_bias.reshape(-1))
    return out.reshape(BATCH, 1)


# speedup vs baseline: 10.6355x; 1.5524x over previous
import jax
import jax.numpy as jnp
from jax import lax
from jax.experimental import pallas as pl
from jax.experimental.pallas import tpu as pltpu
from jax.experimental.pallas import tpu_sc as plsc

NUM_CORES = 2
NUM_SUBCORES = 16
LANES = 16
NUM_WORKERS = NUM_CORES * NUM_SUBCORES

BATCH = 16384
EMBED = 16
NROWS = 1000000
WIN = 128
B_PER_W = BATCH // NUM_WORKERS
CHUNK = 128
NCHUNKS = B_PER_W // CHUNK
NGROUPS = B_PER_W // LANES


def _body(uid_hbm, iid_hbm, uembT_hbm, iembT_hbm, ubT_hbm, ibT_hbm,
          out_hbm, uidx_v, iidx_v, u_buf, i_buf, ub_buf, ib_buf, out_v,
          sem):
    wid = lax.axis_index("s") * NUM_CORES + lax.axis_index("c")
    base = wid * B_PER_W

    for j in range(NCHUNKS):
        pltpu.sync_copy(uid_hbm.at[pl.ds(base + j * CHUNK, CHUNK)],
                        uidx_v.at[j])
        pltpu.sync_copy(iid_hbm.at[pl.ds(base + j * CHUNK, CHUNK)],
                        iidx_v.at[j])

    iota = lax.iota(jnp.int32, LANES)

    def group(g, carry):
        j = g // (CHUNK // LANES)
        k = (g - j * (CHUNK // LANES)) * LANES
        uvec = uidx_v[j, pl.ds(k, LANES)]
        ivec = iidx_v[j, pl.ds(k, LANES)]
        copies = []
        for lane in range(LANES):
            ub_ = pl.multiple_of((uvec[lane] >> 7) << 7, WIN)
            ib_ = pl.multiple_of((ivec[lane] >> 7) << 7, WIN)
            copies.append(pltpu.async_copy(
                uembT_hbm.at[:, pl.ds(ub_, WIN)],
                u_buf.at[:, pl.ds(lane * WIN, WIN)], sem))
            copies.append(pltpu.async_copy(
                iembT_hbm.at[:, pl.ds(ib_, WIN)],
                i_buf.at[:, pl.ds(lane * WIN, WIN)], sem))
            copies.append(pltpu.async_copy(
                ubT_hbm.at[:, pl.ds(ub_, WIN)],
                ub_buf.at[:, pl.ds(lane * WIN, WIN)], sem))
            copies.append(pltpu.async_copy(
                ibT_hbm.at[:, pl.ds(ib_, WIN)],
                ib_buf.at[:, pl.ds(lane * WIN, WIN)], sem))
        for c in copies:
            c.wait()

        lane_u = iota * WIN + (uvec & (WIN - 1))
        lane_i = iota * WIN + (ivec & (WIN - 1))
        acc = jnp.zeros((LANES,), jnp.float32)
        for e in range(EMBED):
            ev = jnp.full((LANES,), e, jnp.int32)
            u = plsc.load_gather(u_buf, [ev, lane_u])
            v = plsc.load_gather(i_buf, [ev, lane_i])
            acc = acc + u * v
        zv = jnp.zeros((LANES,), jnp.int32)
        ubv = plsc.load_gather(ub_buf, [zv, lane_u])
        ibv = plsc.load_gather(ib_buf, [zv, lane_i])
        x = acc + ubv + ibv
        x = jnp.maximum(x, 0.0)
        e2 = jnp.exp(-2.0 * x)
        out_v[pl.ds(g * LANES, LANES)] = (1.0 - e2) / (1.0 + e2)
        return carry

    lax.fori_loop(0, NGROUPS, group, 0)

    pltpu.sync_copy(out_v, out_hbm.at[pl.ds(base, B_PER_W)])


_sc_kernel = pl.kernel(
    _body,
    out_type=jax.ShapeDtypeStruct((BATCH,), jnp.float32),
    mesh=plsc.VectorSubcoreMesh(core_axis_name="c", subcore_axis_name="s"),
    compiler_params=pltpu.CompilerParams(
        needs_layout_passes=False, use_tc_tiling_on_sc=True),
    scratch_types=[
        pltpu.VMEM((NCHUNKS, CHUNK), jnp.int32),
        pltpu.VMEM((NCHUNKS, CHUNK), jnp.int32),
        pltpu.VMEM((EMBED, LANES * WIN), jnp.float32),
        pltpu.VMEM((EMBED, LANES * WIN), jnp.float32),
        pltpu.VMEM((1, LANES * WIN), jnp.float32),
        pltpu.VMEM((1, LANES * WIN), jnp.float32),
        pltpu.VMEM((B_PER_W,), jnp.float32),
        pltpu.SemaphoreType.DMA,
    ],
)


@jax.jit
def kernel(inputs, user_emb, item_emb, user_bias, item_bias):
    uid = inputs[:, 0].astype(jnp.int32)
    iid = inputs[:, 1].astype(jnp.int32)
    out = _sc_kernel(uid, iid, user_emb.T, item_emb.T,
                     user_bias.T, item_bias.T)
    return out.reshape(BATCH, 1)


# speedup vs baseline: 10.8021x; 1.0157x over previous
import jax
import jax.numpy as jnp
from jax import lax
from jax.experimental import pallas as pl
from jax.experimental.pallas import tpu as pltpu
from jax.experimental.pallas import tpu_sc as plsc

NUM_CORES = 2
NUM_SUBCORES = 16
LANES = 16
NUM_WORKERS = NUM_CORES * NUM_SUBCORES

BATCH = 16384
EMBED = 16
NROWS = 1000000
WIN = 128
B_PER_W = BATCH // NUM_WORKERS
CHUNK = 128
NCHUNKS = B_PER_W // CHUNK
NGROUPS = B_PER_W // LANES


def _body(uid_hbm, iid_hbm, uembT_hbm, iembT_hbm, ubT_hbm, ibT_hbm,
          out_hbm, uidx_v, iidx_v, u_buf, i_buf, ub_buf, ib_buf, out_v,
          sem):
    wid = lax.axis_index("s") * NUM_CORES + lax.axis_index("c")
    base = wid * B_PER_W

    for j in range(NCHUNKS):
        pltpu.sync_copy(uid_hbm.at[pl.ds(base + j * CHUNK, CHUNK)],
                        uidx_v.at[j])
        pltpu.sync_copy(iid_hbm.at[pl.ds(base + j * CHUNK, CHUNK)],
                        iidx_v.at[j])

    iota = lax.iota(jnp.int32, LANES)

    def group(g, carry):
        j = g // (CHUNK // LANES)
        k = (g - j * (CHUNK // LANES)) * LANES
        uvec = uidx_v[j, pl.ds(k, LANES)]
        ivec = iidx_v[j, pl.ds(k, LANES)]
        for lane in range(LANES):
            ub_ = pl.multiple_of((uvec[lane] >> 7) << 7, WIN)
            ib_ = pl.multiple_of((ivec[lane] >> 7) << 7, WIN)
            pltpu.async_copy(
                uembT_hbm.at[:, pl.ds(ub_, WIN)],
                u_buf.at[:, pl.ds(lane * WIN, WIN)], sem)
            pltpu.async_copy(
                iembT_hbm.at[:, pl.ds(ib_, WIN)],
                i_buf.at[:, pl.ds(lane * WIN, WIN)], sem)
            pltpu.async_copy(
                ubT_hbm.at[:, pl.ds(ub_, WIN)],
                ub_buf.at[:, pl.ds(lane * WIN, WIN)], sem)
            pltpu.async_copy(
                ibT_hbm.at[:, pl.ds(ib_, WIN)],
                ib_buf.at[:, pl.ds(lane * WIN, WIN)], sem)
        pltpu.make_async_copy(
            uembT_hbm.at[:, pl.ds(0, LANES * WIN)], u_buf, sem).wait()
        pltpu.make_async_copy(
            iembT_hbm.at[:, pl.ds(0, LANES * WIN)], i_buf, sem).wait()
        pltpu.make_async_copy(
            ubT_hbm.at[:, pl.ds(0, LANES * WIN)], ub_buf, sem).wait()
        pltpu.make_async_copy(
            ibT_hbm.at[:, pl.ds(0, LANES * WIN)], ib_buf, sem).wait()

        lane_u = iota * WIN + (uvec & (WIN - 1))
        lane_i = iota * WIN + (ivec & (WIN - 1))
        acc = jnp.zeros((LANES,), jnp.float32)
        for e in range(EMBED):
            ev = jnp.full((LANES,), e, jnp.int32)
            u = plsc.load_gather(u_buf, [ev, lane_u])
            v = plsc.load_gather(i_buf, [ev, lane_i])
            acc = acc + u * v
        zv = jnp.zeros((LANES,), jnp.int32)
        ubv = plsc.load_gather(ub_buf, [zv, lane_u])
        ibv = plsc.load_gather(ib_buf, [zv, lane_i])
        x = acc + ubv + ibv
        x = jnp.maximum(x, 0.0)
        e2 = jnp.exp(-2.0 * x)
        out_v[pl.ds(g * LANES, LANES)] = (1.0 - e2) / (1.0 + e2)
        return carry

    lax.fori_loop(0, NGROUPS, group, 0)

    pltpu.sync_copy(out_v, out_hbm.at[pl.ds(base, B_PER_W)])


_sc_kernel = pl.kernel(
    _body,
    out_type=jax.ShapeDtypeStruct((BATCH,), jnp.float32),
    mesh=plsc.VectorSubcoreMesh(core_axis_name="c", subcore_axis_name="s"),
    compiler_params=pltpu.CompilerParams(
        needs_layout_passes=False, use_tc_tiling_on_sc=True),
    scratch_types=[
        pltpu.VMEM((NCHUNKS, CHUNK), jnp.int32),
        pltpu.VMEM((NCHUNKS, CHUNK), jnp.int32),
        pltpu.VMEM((EMBED, LANES * WIN), jnp.float32),
        pltpu.VMEM((EMBED, LANES * WIN), jnp.float32),
        pltpu.VMEM((1, LANES * WIN), jnp.float32),
        pltpu.VMEM((1, LANES * WIN), jnp.float32),
        pltpu.VMEM((B_PER_W,), jnp.float32),
        pltpu.SemaphoreType.DMA,
    ],
)


@jax.jit
def kernel(inputs, user_emb, item_emb, user_bias, item_bias):
    uid = inputs[:, 0].astype(jnp.int32)
    iid = inputs[:, 1].astype(jnp.int32)
    out = _sc_kernel(uid, iid, user_emb.T, item_emb.T,
                     user_bias.T, item_bias.T)
    return out.reshape(BATCH, 1)
